# baseline (speedup 1.0000x reference)
import functools

import jax
import jax.numpy as jnp
from jax import lax
from jax.experimental import pallas as pl
from jax.experimental.pallas import tpu as pltpu
from jax.experimental.pallas import tpu_sc as plsc

N = 10000
NP = 10240
D = 128
K = 128
E = 320000

CE = 128
CE2 = 64
CH = 80
SB = 16
EP = 32 * CH * CE
NSUB = 16
ROWS_PER_SUB = NP // NSUB


def _dot(a, b):
    return lax.dot_general(a, b, (((1,), (0,)), ((), ())),
                           preferred_element_type=jnp.float32)


def _dtt(a, b):
    return lax.dot_general(a, b, (((1,), (1,)), ((), ())),
                           preferred_element_type=jnp.float32)


def _k1_body(from_partials, x_ref, maskp_ref, srow_ref, q_ref,
             wu, uu, wr, ur, wh, uh, bu, br, bh,
             q_out, y_out, xscr):
    if from_partials:
        X = jnp.maximum(x_ref[0] + x_ref[1], 0.0)
    else:
        X = x_ref[...]
    xscr[...] = X

    srow = srow_ref[...]
    n = jnp.sqrt(jnp.sum(srow * srow))
    sc = (_dtt(srow, X) / n) + maskp_ref[...]
    lin1 = lax.broadcasted_iota(jnp.int32, (1, NP), 1)
    krows = lax.broadcasted_iota(jnp.int32, (K, 1), 0)

    def it(k, carry):
        s, sel = carry
        m = jnp.max(s)
        pos = jnp.min(jnp.where(s == m, lin1, jnp.int32(2 ** 30)))
        row = xscr[pl.ds(pos, 1), :]
        sel = jnp.where(krows == k, row * jnp.tanh(m), sel)
        return jnp.where(lin1 == pos, -jnp.inf, s), sel

    _, sel = lax.fori_loop(0, K, it,
                           (sc, jnp.zeros((K, D), jnp.float32)))

    q = q_ref[...]
    upd = jax.nn.sigmoid(_dtt(wu[...], sel) + _dot(uu[...], q) + bu[...])
    rst = jax.nn.sigmoid(_dtt(wr[...], sel) + _dot(ur[...], q) + br[...])
    hcap = jnp.tanh(_dtt(wh[...], sel) + _dot(uh[...], rst * q) + bh[...])
    qn = (1.0 - upd) * q + upd * hcap
    q_out[...] = qn
    y_out[...] = _dot(xscr[...], qn)


def _make_k1(from_partials):
    return pl.pallas_call(
        functools.partial(_k1_body, from_partials),
        out_shape=[jax.ShapeDtypeStruct((D, K), jnp.float32),
                   jax.ShapeDtypeStruct((NP, K), jnp.float32)],
        scratch_shapes=[pltpu.VMEM((NP, D), jnp.float32)],
    )


def _spmm_sc(y0, y1, src_0, dst_0, w_0, src_1, dst_1, w_1):
    mesh = plsc.VectorSubcoreMesh(core_axis_name="c", subcore_axis_name="s")

    @functools.partial(
        pl.kernel,
        out_type=jax.ShapeDtypeStruct((2, 2, NP, D), jnp.float32),
        mesh=mesh,
        compiler_params=pltpu.CompilerParams(needs_layout_passes=False),
        scratch_types=[
            pltpu.VMEM((SB, CE), jnp.int32),
            pltpu.VMEM((2 * SB, CE2), jnp.int32),
            pltpu.VMEM((SB, CE), jnp.float32),
            pltpu.VMEM((CE2, D), jnp.float32),
            pltpu.VMEM((CE2, D), jnp.float32),
            pltpu.VMEM_SHARED((NP, D), jnp.float32),
            pltpu.SemaphoreType.DMA,
            pltpu.SemaphoreType.DMA,
        ],
    )
    def k(y0_hbm, y1_hbm, src0_hbm, dst0_hbm, w0_hbm,
          src1_hbm, dst1_hbm, w1_hbm, out_hbm, src_v, dst_v, w_v,
          gbuf0, gbuf1, acc, sem0, sem1):
        c = lax.axis_index("c")
        s = lax.axis_index("s")
        wid = c * NSUB + s

        def zero_acc_slice():
            @pl.loop(0, CE2)
            def _(i):
                for q in range(D // 16):
                    gbuf0[i, pl.ds(q * 16, 16)] = jnp.zeros((16,),
                                                            jnp.float32)

            @pl.loop(0, ROWS_PER_SUB, step=CE2)
            def _(r0):
                pltpu.sync_copy(gbuf0,
                                acc.at[pl.ds(s * ROWS_PER_SUB + r0, CE2)])

        def run_edges(y_hbm, src_hbm, dst_hbm, w_hbm):
            @pl.loop(0, CH, step=SB)
            def _(j0):
                pltpu.sync_copy(src_hbm.at[wid, pl.ds(j0, SB)], src_v)
                pltpu.sync_copy(dst_hbm.at[wid, pl.ds(2 * j0, 2 * SB)],
                                dst_v)
                pltpu.sync_copy(w_hbm.at[wid, pl.ds(j0, SB)], w_v)
                pltpu.async_copy(y_hbm.at[src_v.at[0, pl.ds(0, CE2)]],
                                 gbuf0, sem0)

                @pl.loop(0, SB)
                def _(j):
                    for h, gb, sem in ((0, gbuf0, sem0), (1, gbuf1, sem1)):
                        pltpu.make_async_copy(
                            y_hbm.at[src_v.at[j, pl.ds(h * CE2, CE2)]],
                            gb, sem).wait()
                        if h == 0:
                            pltpu.async_copy(
                                y_hbm.at[src_v.at[j, pl.ds(CE2, CE2)]],
                                gbuf1, sem1)
                        else:
                            @pl.when(j < SB - 1)
                            def _():
                                pltpu.async_copy(
                                    y_hbm.at[src_v.at[j + 1,
                                                      pl.ds(0, CE2)]],
                                    gbuf0, sem0)

                        @pl.loop(0, CE2, step=4)
                        def _(r0):
                            for rr in range(4):
                                r = r0 + rr
                                wb = plsc.load_gather(
                                    w_v, [jnp.full((16,), j, jnp.int32),
                                          jnp.full((16,), h * CE2 + r,
                                                   jnp.int32)])
                                for q in range(D // 16):
                                    gb[r, pl.ds(q * 16, 16)] = (
                                        gb[r, pl.ds(q * 16, 16)] * wb)

                        pltpu.sync_copy(gb, acc.at[dst_v.at[2 * j + h]],
                                        add=True)

        def copy_out(t):
            @pl.loop(0, ROWS_PER_SUB, step=CE)
            def _(r0):
                pltpu.sync_copy(
                    acc.at[pl.ds(s * ROWS_PER_SUB + r0, CE)],
                    out_hbm.at[t, c, pl.ds(s * ROWS_PER_SUB + r0, CE)])

        zero_acc_slice()
        plsc.subcore_barrier()
        run_edges(y0_hbm, src0_hbm, dst0_hbm, w0_hbm)
        plsc.subcore_barrier()
        copy_out(0)
        zero_acc_slice()
        plsc.subcore_barrier()
        run_edges(y1_hbm, src1_hbm, dst1_hbm, w1_hbm)
        plsc.subcore_barrier()
        copy_out(1)

    return k(y0, y1, src_0, dst_0, w_0, src_1, dst_1, w_1)


def _k4_body(p0_ref, p1_ref, o_ref):
    o_ref[0] = jnp.maximum(p0_ref[0] + p0_ref[1], 0.0)
    o_ref[1] = jnp.maximum(p1_ref[0] + p1_ref[1], 0.0)


_BK4 = 400


def _k4(p0, p1):
    return pl.pallas_call(
        _k4_body,
        grid=(N // _BK4,),
        in_specs=[pl.BlockSpec((2, _BK4, D), lambda i: (0, i, 0)),
                  pl.BlockSpec((2, _BK4, D), lambda i: (0, i, 0))],
        out_specs=pl.BlockSpec((2, _BK4, D), lambda i: (0, i, 0)),
        out_shape=jax.ShapeDtypeStruct((2, N, D), jnp.float32),
    )(p0, p1)


def kernel(x_t0, x_t1, edge_index_t0, edge_index_t1, edge_weight_t0,
           edge_weight_t1, mask_t0, mask_t1, W_init_0, W_init_1,
           scorer_l0, Wu_l0, Uu_l0, bu_l0, Wr_l0, Ur_l0, br_l0, Wh_l0, Uh_l0,
           bh_l0, scorer_l1, Wu_l1, Uu_l1, bu_l1, Wr_l1, Ur_l1, br_l1, Wh_l1,
           Uh_l1, bh_l1):
    f32 = jnp.float32
    pad = NP - N
    xp0 = jnp.pad(x_t0, ((0, pad), (0, 0)))
    xp1 = jnp.pad(x_t1, ((0, pad), (0, 0)))
    negbig = jnp.full((pad,), -1e30, f32)
    mp0 = jnp.concatenate([mask_t0[:, 0], negbig]).reshape(1, NP)
    mp1 = jnp.concatenate([mask_t1[:, 0], negbig]).reshape(1, NP)

    epad = EP - E
    idxpad = jnp.full((epad,), N, jnp.int32)
    wpad = jnp.zeros((epad,), f32)
    src0 = jnp.concatenate([edge_index_t0[0], idxpad]).reshape(32, CH, CE)
    dst0 = jnp.concatenate([edge_index_t0[1], idxpad]).reshape(32, 2 * CH, CE2)
    w0 = jnp.concatenate([edge_weight_t0, wpad]).reshape(32, CH, CE)
    src1 = jnp.concatenate([edge_index_t1[0], idxpad]).reshape(32, CH, CE)
    dst1 = jnp.concatenate([edge_index_t1[1], idxpad]).reshape(32, 2 * CH, CE2)
    w1 = jnp.concatenate([edge_weight_t1, wpad]).reshape(32, CH, CE)

    params0 = (Wu_l0, Uu_l0, Wr_l0, Ur_l0, Wh_l0, Uh_l0, bu_l0, br_l0, bh_l0)
    params1 = (Wu_l1, Uu_l1, Wr_l1, Ur_l1, Wh_l1, Uh_l1, bu_l1, br_l1, bh_l1)
    srow0 = scorer_l0.reshape(1, 128)
    srow1 = scorer_l1.reshape(1, 128)

    k1a = _make_k1(False)
    k1b = _make_k1(True)

    q, y0 = k1a(xp0, mp0, srow0, W_init_0, *params0)
    q, y1 = k1a(xp1, mp1, srow0, q, *params0)
    p = _spmm_sc(y0, y1, src0, dst0, w0, src1, dst1, w1)

    q, y0 = k1b(p[0], mp0, srow1, W_init_1, *params1)
    q, y1 = k1b(p[1], mp1, srow1, q, *params1)
    p = _spmm_sc(y0, y1, src0, dst0, w0, src1, dst1, w1)

    return _k4(p[0], p[1])


# speedup vs baseline: 1.2226x; 1.2226x over previous
import functools

import jax
import jax.numpy as jnp
from jax import lax
from jax.experimental import pallas as pl
from jax.experimental.pallas import tpu as pltpu
from jax.experimental.pallas import tpu_sc as plsc

N = 10000
NP = 10240
D = 128
K = 128
E = 320000

CE = 128
CE2 = 64
CH = 80
SB = 16
EP = 32 * CH * CE
NSUB = 16
ROWS_PER_SUB = NP // NSUB


def _dot(a, b):
    return lax.dot_general(a, b, (((1,), (0,)), ((), ())),
                           preferred_element_type=jnp.float32)


def _dtt(a, b):
    return lax.dot_general(a, b, (((1,), (1,)), ((), ())),
                           preferred_element_type=jnp.float32)


def _k1_body(from_partials, x_ref, maskp_ref, srow_ref, q_ref,
             wu, uu, wr, ur, wh, uh, bu, br, bh,
             q_out, y_out, xscr):
    if from_partials:
        X = jnp.maximum(x_ref[0] + x_ref[1], 0.0)
    else:
        X = x_ref[...]
    xscr[...] = X

    srow = srow_ref[...]
    n = jnp.sqrt(jnp.sum(srow * srow))
    sc = (_dtt(srow, X) / n) + maskp_ref[...]
    lin1 = lax.broadcasted_iota(jnp.int32, (1, NP), 1)
    krows = lax.broadcasted_iota(jnp.int32, (K, 1), 0)

    def it(k, carry):
        s, sel = carry
        m = jnp.max(s)
        pos = jnp.min(jnp.where(s == m, lin1, jnp.int32(2 ** 30)))
        row = xscr[pl.ds(pos, 1), :]
        sel = jnp.where(krows == k, row * jnp.tanh(m), sel)
        return jnp.where(lin1 == pos, -jnp.inf, s), sel

    _, sel = lax.fori_loop(0, K, it,
                           (sc, jnp.zeros((K, D), jnp.float32)))

    q = q_ref[...]
    upd = jax.nn.sigmoid(_dtt(wu[...], sel) + _dot(uu[...], q) + bu[...])
    rst = jax.nn.sigmoid(_dtt(wr[...], sel) + _dot(ur[...], q) + br[...])
    hcap = jnp.tanh(_dtt(wh[...], sel) + _dot(uh[...], rst * q) + bh[...])
    qn = (1.0 - upd) * q + upd * hcap
    q_out[...] = qn
    y_out[...] = _dot(xscr[...], qn)


def _make_k1(from_partials):
    return pl.pallas_call(
        functools.partial(_k1_body, from_partials),
        out_shape=[jax.ShapeDtypeStruct((D, K), jnp.float32),
                   jax.ShapeDtypeStruct((NP, K), jnp.float32)],
        scratch_shapes=[pltpu.VMEM((NP, D), jnp.float32)],
    )


def _spmm_sc(y, src2, dst2, w2):
    mesh = plsc.VectorSubcoreMesh(core_axis_name="c", subcore_axis_name="s")

    @functools.partial(
        pl.kernel,
        out_type=jax.ShapeDtypeStruct((2, NP, D), jnp.float32),
        mesh=mesh,
        compiler_params=pltpu.CompilerParams(needs_layout_passes=False),
        scratch_types=[
            pltpu.VMEM((SB, CE), jnp.int32),
            pltpu.VMEM((2 * SB, CE2), jnp.int32),
            pltpu.VMEM((SB, CE), jnp.float32),
            pltpu.VMEM((CE2, D), jnp.float32),
            pltpu.VMEM((CE2, D), jnp.float32),
            pltpu.VMEM_SHARED((NP, D), jnp.float32),
            pltpu.SemaphoreType.DMA,
            pltpu.SemaphoreType.DMA,
        ],
    )
    def k(y_hbm, src_hbm, dst_hbm, w_hbm, out_hbm, src_v, dst_v, w_v,
          gbuf0, gbuf1, acc, sem0, sem1):
        c = lax.axis_index("c")
        s = lax.axis_index("s")
        wid = c * NSUB + s

        @pl.loop(0, CE2)
        def _(i):
            for q in range(D // 16):
                gbuf0[i, pl.ds(q * 16, 16)] = jnp.zeros((16,), jnp.float32)

        @pl.loop(0, ROWS_PER_SUB, step=CE2)
        def _(r0):
            pltpu.sync_copy(gbuf0, acc.at[pl.ds(s * ROWS_PER_SUB + r0, CE2)])

        plsc.subcore_barrier()

        @pl.loop(0, CH, step=SB)
        def _(j0):
            pltpu.sync_copy(src_hbm.at[wid, pl.ds(j0, SB)], src_v)
            pltpu.sync_copy(dst_hbm.at[wid, pl.ds(2 * j0, 2 * SB)], dst_v)
            pltpu.sync_copy(w_hbm.at[wid, pl.ds(j0, SB)], w_v)
            pltpu.async_copy(y_hbm.at[src_v.at[0, pl.ds(0, CE2)]],
                             gbuf0, sem0)

            @pl.loop(0, SB)
            def _(j):
                for h, gb, sem in ((0, gbuf0, sem0), (1, gbuf1, sem1)):
                    pltpu.make_async_copy(
                        y_hbm.at[src_v.at[j, pl.ds(h * CE2, CE2)]],
                        gb, sem).wait()
                    if h == 0:
                        pltpu.async_copy(
                            y_hbm.at[src_v.at[j, pl.ds(CE2, CE2)]],
                            gbuf1, sem1)
                    else:
                        @pl.when(j < SB - 1)
                        def _():
                            pltpu.async_copy(
                                y_hbm.at[src_v.at[j + 1, pl.ds(0, CE2)]],
                                gbuf0, sem0)

                    @pl.loop(0, CE2, step=4)
                    def _(r0):
                        for rr in range(4):
                            r = r0 + rr
                            wb = plsc.load_gather(
                                w_v, [jnp.full((16,), j, jnp.int32),
                                      jnp.full((16,), h * CE2 + r,
                                               jnp.int32)])
                            for q in range(D // 16):
                                gb[r, pl.ds(q * 16, 16)] = (
                                    gb[r, pl.ds(q * 16, 16)] * wb)

                    pltpu.sync_copy(gb, acc.at[dst_v.at[2 * j + h]],
                                    add=True)

        plsc.subcore_barrier()

        @pl.loop(0, ROWS_PER_SUB, step=CE)
        def _(r0):
            pltpu.sync_copy(acc.at[pl.ds(s * ROWS_PER_SUB + r0, CE)],
                            out_hbm.at[c, pl.ds(s * ROWS_PER_SUB + r0, CE)])

    return k(y, src2, dst2, w2)


def _k4_body(p0_ref, p1_ref, o_ref):
    o_ref[0] = jnp.maximum(p0_ref[0] + p0_ref[1], 0.0)
    o_ref[1] = jnp.maximum(p1_ref[0] + p1_ref[1], 0.0)


_BK4 = 400


def _k4(p0, p1):
    return pl.pallas_call(
        _k4_body,
        grid=(N // _BK4,),
        in_specs=[pl.BlockSpec((2, _BK4, D), lambda i: (0, i, 0)),
                  pl.BlockSpec((2, _BK4, D), lambda i: (0, i, 0))],
        out_specs=pl.BlockSpec((2, _BK4, D), lambda i: (0, i, 0)),
        out_shape=jax.ShapeDtypeStruct((2, N, D), jnp.float32),
    )(p0, p1)


def kernel(x_t0, x_t1, edge_index_t0, edge_index_t1, edge_weight_t0,
           edge_weight_t1, mask_t0, mask_t1, W_init_0, W_init_1,
           scorer_l0, Wu_l0, Uu_l0, bu_l0, Wr_l0, Ur_l0, br_l0, Wh_l0, Uh_l0,
           bh_l0, scorer_l1, Wu_l1, Uu_l1, bu_l1, Wr_l1, Ur_l1, br_l1, Wh_l1,
           Uh_l1, bh_l1):
    f32 = jnp.float32
    pad = NP - N
    xp0 = jnp.pad(x_t0, ((0, pad), (0, 0)))
    xp1 = jnp.pad(x_t1, ((0, pad), (0, 0)))
    negbig = jnp.full((pad,), -1e30, f32)
    mp0 = jnp.concatenate([mask_t0[:, 0], negbig]).reshape(1, NP)
    mp1 = jnp.concatenate([mask_t1[:, 0], negbig]).reshape(1, NP)

    epad = EP - E
    idxpad = jnp.full((epad,), N, jnp.int32)
    wpad = jnp.zeros((epad,), f32)
    src0 = jnp.concatenate([edge_index_t0[0], idxpad]).reshape(32, CH, CE)
    dst0 = jnp.concatenate([edge_index_t0[1], idxpad]).reshape(32, 2 * CH, CE2)
    w0 = jnp.concatenate([edge_weight_t0, wpad]).reshape(32, CH, CE)
    src1 = jnp.concatenate([edge_index_t1[0], idxpad]).reshape(32, CH, CE)
    dst1 = jnp.concatenate([edge_index_t1[1], idxpad]).reshape(32, 2 * CH, CE2)
    w1 = jnp.concatenate([edge_weight_t1, wpad]).reshape(32, CH, CE)

    params0 = (Wu_l0, Uu_l0, Wr_l0, Ur_l0, Wh_l0, Uh_l0, bu_l0, br_l0, bh_l0)
    params1 = (Wu_l1, Uu_l1, Wr_l1, Ur_l1, Wh_l1, Uh_l1, bu_l1, br_l1, bh_l1)
    srow0 = scorer_l0.reshape(1, 128)
    srow1 = scorer_l1.reshape(1, 128)

    k1a = _make_k1(False)
    k1b = _make_k1(True)

    q, y0 = k1a(xp0, mp0, srow0, W_init_0, *params0)
    q, y1 = k1a(xp1, mp1, srow0, q, *params0)
    p0 = _spmm_sc(y0, src0, dst0, w0)
    p1 = _spmm_sc(y1, src1, dst1, w1)

    q, y0 = k1b(p0, mp0, srow1, W_init_1, *params1)
    q, y1 = k1b(p1, mp1, srow1, q, *params1)
    p0 = _spmm_sc(y0, src0, dst0, w0)
    p1 = _spmm_sc(y1, src1, dst1, w1)

    return _k4(p0, p1)
